# baseline (speedup 1.0000x reference)
import jax
import jax.numpy as jnp
from jax import lax
from jax.experimental import pallas as pl
from jax.experimental.pallas import tpu as pltpu
from jax.experimental.pallas import tpu_sc as plsc

NTOT = 1536
NATM = 384
NOPS = 8
NS = 4
NU = NS * (NOPS + 1)
NW = 32
R2 = 0.4 * 0.4


def _floor(x):
    t = x.astype(jnp.int32).astype(jnp.float32)
    return jnp.where(x < t, t - 1.0, t)


def _rb16(x):
    t1 = x * jnp.float32(65537.0)
    return t1 - (t1 - x)


def _sc_body(xs_hbm, ti_hbm, tf_hbm, out_hbm, xs_v, f1_v, ti_v, tf_v, st_v):
    cid = lax.axis_index("c")
    sid = lax.axis_index("s")
    w = sid * 2 + cid
    pltpu.sync_copy(xs_hbm, xs_v)
    pltpu.sync_copy(ti_hbm, ti_v)
    pltpu.sync_copy(tf_hbm, tf_v)
    lane = lax.broadcasted_iota(jnp.int32, (16,), 0)
    zero16 = jnp.zeros((16,), jnp.float32)

    def unit_body(u, k):
        tiu = ti_v[pl.ds(u * 16, 16)]
        natm = tiu[0]
        fa = tiu[1]
        ca = tiu[2]
        nch = tiu[3]
        ci = jnp.remainder(w - k, NW)

        @pl.when(ci < nch)
        def _():
            tfu = _rb16(tf_v[pl.ds(u * 16, 16)])

            @pl.when(u < NS)
            def _():
                def stage0(ch, carry):
                    gb = (ca + ch) * 16
                    for c in range(3):
                        f1_v[pl.ds(c * NTOT + gb, 16)] = \
                            xs_v[pl.ds(c * NTOT + gb, 16)]
                    return carry

                lax.fori_loop(0, nch, stage0, 0)

            @pl.when(u >= NS)
            def _():
                def stage1(ch, carry):
                    gb = (ca + ch) * 16
                    x = _rb16(xs_v[pl.ds(0 * NTOT + gb, 16)])
                    y = _rb16(xs_v[pl.ds(1 * NTOT + gb, 16)])
                    z = _rb16(xs_v[pl.ds(2 * NTOT + gb, 16)])
                    for c in range(3):
                        t = tfu[3 * c + 0] * x + tfu[3 * c + 1] * y \
                            + tfu[3 * c + 2] * z
                        f1_v[pl.ds(c * NTOT + gb, 16)] = t - _floor(t)
                    return carry

                lax.fori_loop(0, nch, stage1, 0)

            ib = (ca + ci) * 16
            ux16 = f1_v[pl.ds(0 * NTOT + ib, 16)]
            uy16 = f1_v[pl.ds(1 * NTOT + ib, 16)]
            uz16 = f1_v[pl.ds(2 * NTOT + ib, 16)]
            accs = (zero16, zero16, zero16)
            for g in range(2):
                rows = []
                for l in range(8):
                    li = g * 8 + l
                    gi = ib + li
                    ok_row = (gi >= fa) & (gi < fa + natm)
                    lim = jnp.where(ok_row, natm, 0).astype(jnp.uint32)
                    rows.append((jnp.broadcast_to(ux16[li], (16,)),
                                 jnp.broadcast_to(uy16[li], (16,)),
                                 jnp.broadcast_to(uz16[li], (16,)),
                                 lim))

                def cj_body(cj, accs2):
                    ax, ay, az = accs2
                    jb = (ca + cj) * 16
                    jd = (jb + lane - fa).astype(jnp.uint32)
                    xj = f1_v[pl.ds(0 * NTOT + jb, 16)]
                    yj = f1_v[pl.ds(1 * NTOT + jb, 16)]
                    zj = f1_v[pl.ds(2 * NTOT + jb, 16)]
                    for ux, uy, uz, lim in rows:
                        adx = jnp.abs(xj - ux)
                        ady = jnp.abs(yj - uy)
                        adz = jnp.abs(zj - uz)
                        tx = jnp.minimum(adx, 1.0 - adx)
                        ty = jnp.minimum(ady, 1.0 - ady)
                        tz = jnp.minimum(adz, 1.0 - adz)
                        sx = tx * tx
                        sy = ty * ty
                        sz = tz * tz
                        d2 = sx + sy + sz
                        ok = (d2 <= R2) & (jd < lim)
                        ax = ax + jnp.where(ok, sx, zero16)
                        ay = ay + jnp.where(ok, sy, zero16)
                        az = az + jnp.where(ok, sz, zero16)
                    return ax, ay, az

                accs = lax.fori_loop(0, nch, cj_body, accs)
            ax, ay, az = accs
            st_v[pl.ds(0, 16)] = ax
            st_v[pl.ds(16, 16)] = ay
            st_v[pl.ds(32, 16)] = az
            pltpu.sync_copy(st_v, out_hbm.at[u, w])

        return k + nch

    lax.fori_loop(0, NU, unit_body, jnp.int32(0))


def _prep_body(natm_ref, nopr_ref, ops_ref, ti_ref, tf_ref, w_ref, m_ref):
    natms = [natm_ref[i] for i in range(NS)]
    noprs = [nopr_ref[i] for i in range(NS)]
    fa = []
    oa = []
    run_a = jnp.int32(0)
    run_o = jnp.int32(0)
    for s in range(NS):
        fa.append(run_a)
        oa.append(run_o)
        run_a = run_a + natms[s]
        run_o = run_o + noprs[s]
    k_run = jnp.int32(0)
    for u in range(NU):
        s = u % NS
        slot = u // NS
        if slot == 0:
            active = jnp.bool_(True)
            for a in range(3):
                for b in range(3):
                    tf_ref[u, 3 * a + b] = jnp.float32(1.0 if a == b else 0.0)
        else:
            active = (slot - 1) < noprs[s]
            opidx = jnp.clip(oa[s] + (slot - 1), 0, 31)
            for a in range(3):
                for b in range(3):
                    tf_ref[u, 3 * a + b] = ops_ref[opidx, a, b]
        natm_eff = jnp.where(active, natms[s], 0)
        ca = fa[s] // 16
        nch = jnp.where(natm_eff > 0,
                        (fa[s] + natm_eff - 1) // 16 - ca + 1, 0)
        ti_ref[u, 0] = natm_eff
        ti_ref[u, 1] = fa[s]
        ti_ref[u, 2] = ca
        ti_ref[u, 3] = nch
        lo = jnp.remainder(k_run, NW)
        for w in range(NW):
            ci = jnp.remainder(w - lo, NW)
            m_ref[u, w, 0] = jnp.where(ci < nch, jnp.float32(1.0),
                                       jnp.float32(0.0))
        k_run = k_run + nch
    for j in range(NOPS):
        for s in range(NS):
            inv = 1.0 / (jnp.maximum(noprs[s], 1).astype(jnp.float32) * NS)
            w_ref[j * NS + s, 0] = jnp.where(j < noprs[s], inv,
                                             jnp.float32(0.0))


def _combine_body(p_ref, m_ref, w_ref, o_ref):
    p = jnp.where(m_ref[...] > 0, p_ref[...], 0.0)
    q = jnp.sum(p, axis=1)
    sx = jnp.sum(q[:, 0:16], axis=1, keepdims=True)
    sy = jnp.sum(q[:, 16:32], axis=1, keepdims=True)
    sz = jnp.sum(q[:, 32:48], axis=1, keepdims=True)
    S = jnp.concatenate([sx, sy, sz], axis=1)
    base = S[0:NS, :]
    rest = S[NS:, :]
    tiled = jnp.concatenate([base] * NOPS, axis=0)
    d = rest - tiled
    n2 = jnp.sum(d * d, axis=1, keepdims=True)
    o_ref[0, 0] = jnp.sum(w_ref[...] * jnp.sqrt(n2))


def kernel(fracs, natms, oprss, noprs):
    natms = natms.reshape(-1).astype(jnp.int32)
    noprs = noprs.reshape(-1).astype(jnp.int32)

    ti, tf, w32, mask3 = pl.pallas_call(
        _prep_body,
        in_specs=[
            pl.BlockSpec(memory_space=pltpu.SMEM),
            pl.BlockSpec(memory_space=pltpu.SMEM),
            pl.BlockSpec(memory_space=pltpu.SMEM),
        ],
        out_specs=(
            pl.BlockSpec(memory_space=pltpu.SMEM),
            pl.BlockSpec(memory_space=pltpu.SMEM),
            pl.BlockSpec(memory_space=pltpu.SMEM),
            pl.BlockSpec(memory_space=pltpu.SMEM),
        ),
        out_shape=(
            jax.ShapeDtypeStruct((NU, 16), jnp.int32),
            jax.ShapeDtypeStruct((NU, 16), jnp.float32),
            jax.ShapeDtypeStruct((NS * NOPS, 1), jnp.float32),
            jax.ShapeDtypeStruct((NU, NW, 1), jnp.float32),
        ),
    )(natms, noprs, oprss)

    mesh = plsc.VectorSubcoreMesh(core_axis_name="c", subcore_axis_name="s")
    partial = pl.kernel(
        _sc_body,
        out_type=jax.ShapeDtypeStruct((NU, NW, 48), jnp.float32),
        mesh=mesh,
        scratch_types=[
            pltpu.VMEM((3 * NTOT,), jnp.float32),
            pltpu.VMEM((3 * NTOT,), jnp.float32),
            pltpu.VMEM((NU * 16,), jnp.int32),
            pltpu.VMEM((NU * 16,), jnp.float32),
            pltpu.VMEM((48,), jnp.float32),
        ],
    )(fracs.T.reshape(-1), ti.reshape(-1), tf.reshape(-1))

    out = pl.pallas_call(
        _combine_body,
        out_specs=pl.BlockSpec(memory_space=pltpu.SMEM),
        out_shape=jax.ShapeDtypeStruct((1, 1), jnp.float32),
    )(partial, mask3, w32)
    return out[0, 0]


# speedup vs baseline: 2.2351x; 2.2351x over previous
import jax
import jax.numpy as jnp
from jax import lax
from jax.experimental import pallas as pl
from jax.experimental.pallas import tpu as pltpu
from jax.experimental.pallas import tpu_sc as plsc

NTOT = 1536
NATM = 384
NOPS = 8
NS = 4
NU = NS * (NOPS + 1)
NW = 32
R2 = 0.4 * 0.4


def _floor(x):
    t = x.astype(jnp.int32).astype(jnp.float32)
    return jnp.where(x < t, t - 1.0, t)


def _rb16(x):
    t1 = x * jnp.float32(65537.0)
    return t1 - (t1 - x)


def _sc_body(xs_hbm, ti_hbm, tf_hbm, out_hbm, xs_v, f1_v, ti_v, tf_v, st_v):
    cid = lax.axis_index("c")
    sid = lax.axis_index("s")
    w = sid * 2 + cid
    pltpu.sync_copy(xs_hbm, xs_v)
    pltpu.sync_copy(ti_hbm, ti_v)
    pltpu.sync_copy(tf_hbm, tf_v)
    lane = lax.broadcasted_iota(jnp.int32, (16,), 0)
    zero16 = jnp.zeros((16,), jnp.float32)

    def unit_body(u, k):
        tiu = ti_v[pl.ds(u * 16, 16)]
        natm = tiu[0]
        fa = tiu[1]
        ca = tiu[2]
        nch = tiu[3]
        ci = jnp.remainder(w - k, NW)

        @pl.when(ci < nch)
        def _():
            tfu = _rb16(tf_v[pl.ds(u * 16, 16)])

            @pl.when(u < NS)
            def _():
                def stage0(ch, carry):
                    gb = (ca + ch) * 16
                    for c in range(3):
                        f1_v[pl.ds(c * NTOT + gb, 16)] = \
                            xs_v[pl.ds(c * NTOT + gb, 16)]
                    return carry

                lax.fori_loop(0, nch, stage0, 0)

            @pl.when(u >= NS)
            def _():
                def stage1(ch, carry):
                    gb = (ca + ch) * 16
                    x = _rb16(xs_v[pl.ds(0 * NTOT + gb, 16)])
                    y = _rb16(xs_v[pl.ds(1 * NTOT + gb, 16)])
                    z = _rb16(xs_v[pl.ds(2 * NTOT + gb, 16)])
                    for c in range(3):
                        t = tfu[3 * c + 0] * x + tfu[3 * c + 1] * y \
                            + tfu[3 * c + 2] * z
                        f1_v[pl.ds(c * NTOT + gb, 16)] = t - _floor(t)
                    return carry

                lax.fori_loop(0, nch, stage1, 0)

            ib = (ca + ci) * 16
            ux16 = f1_v[pl.ds(0 * NTOT + ib, 16)]
            uy16 = f1_v[pl.ds(1 * NTOT + ib, 16)]
            uz16 = f1_v[pl.ds(2 * NTOT + ib, 16)]
            accs = (zero16, zero16, zero16)
            for g in range(2):
                rows = []
                for l in range(8):
                    li = g * 8 + l
                    gi = ib + li
                    ok_row = (gi >= fa) & (gi < fa + natm)
                    lim = jnp.where(ok_row, natm, 0).astype(jnp.uint32)
                    rows.append((jnp.broadcast_to(ux16[li], (16,)),
                                 jnp.broadcast_to(uy16[li], (16,)),
                                 jnp.broadcast_to(uz16[li], (16,)),
                                 lim))

                def cj_body(cj, accs2):
                    ax, ay, az = accs2
                    jb = (ca + cj) * 16
                    jd = (jb + lane - fa).astype(jnp.uint32)
                    xj = f1_v[pl.ds(0 * NTOT + jb, 16)]
                    yj = f1_v[pl.ds(1 * NTOT + jb, 16)]
                    zj = f1_v[pl.ds(2 * NTOT + jb, 16)]
                    for ux, uy, uz, lim in rows:
                        adx = jnp.abs(xj - ux)
                        ady = jnp.abs(yj - uy)
                        adz = jnp.abs(zj - uz)
                        tx = jnp.minimum(adx, 1.0 - adx)
                        ty = jnp.minimum(ady, 1.0 - ady)
                        tz = jnp.minimum(adz, 1.0 - adz)
                        sx = tx * tx
                        sy = ty * ty
                        sz = tz * tz
                        d2 = sx + sy + sz
                        ok = (d2 <= R2) & (jd < lim)
                        ax = ax + jnp.where(ok, sx, zero16)
                        ay = ay + jnp.where(ok, sy, zero16)
                        az = az + jnp.where(ok, sz, zero16)
                    return ax, ay, az

                accs = lax.fori_loop(0, nch, cj_body, accs)
            ax, ay, az = accs
            st_v[pl.ds(0, 16)] = ax
            st_v[pl.ds(16, 16)] = ay
            st_v[pl.ds(32, 16)] = az
            pltpu.sync_copy(st_v, out_hbm.at[u, w])

        return k + nch

    lax.fori_loop(0, NU, unit_body, jnp.int32(0))


def _prep_body(natm_ref, nopr_ref, ops_ref, ti_ref, tf_ref, w_ref):
    natms = [natm_ref[i] for i in range(NS)]
    noprs = [nopr_ref[i] for i in range(NS)]
    fa = []
    oa = []
    run_a = jnp.int32(0)
    run_o = jnp.int32(0)
    for s in range(NS):
        fa.append(run_a)
        oa.append(run_o)
        run_a = run_a + natms[s]
        run_o = run_o + noprs[s]
    k_run = jnp.int32(0)
    for u in range(NU):
        s = u % NS
        slot = u // NS
        if slot == 0:
            active = jnp.bool_(True)
            for a in range(3):
                for b in range(3):
                    tf_ref[u, 3 * a + b] = jnp.float32(1.0 if a == b else 0.0)
        else:
            active = (slot - 1) < noprs[s]
            opidx = jnp.clip(oa[s] + (slot - 1), 0, 31)
            for a in range(3):
                for b in range(3):
                    tf_ref[u, 3 * a + b] = ops_ref[opidx, a, b]
        natm_eff = jnp.where(active, natms[s], 0)
        ca = fa[s] // 16
        nch = jnp.where(natm_eff > 0,
                        (fa[s] + natm_eff - 1) // 16 - ca + 1, 0)
        ti_ref[u, 0] = natm_eff
        ti_ref[u, 1] = fa[s]
        ti_ref[u, 2] = ca
        ti_ref[u, 3] = nch
        ti_ref[u, 4] = k_run
        k_run = k_run + nch
        k_run = jnp.where(k_run >= NW, k_run - NW, k_run)
    for j in range(NOPS):
        for s in range(NS):
            inv = 1.0 / (jnp.maximum(noprs[s], 1).astype(jnp.float32) * NS)
            w_ref[j * NS + s, 0] = jnp.where(j < noprs[s], inv,
                                             jnp.float32(0.0))


def _combine_body(p_ref, t3_ref, w_ref, o_ref):
    nch3 = t3_ref[:, 3:4, :]
    lo3 = t3_ref[:, 4:5, :]
    ww3 = jax.lax.broadcasted_iota(jnp.int32, (NU, NW, 1), 1)
    ci = ww3 - lo3
    ci = jnp.where(ci < 0, ci + NW, ci)
    owned = ci < nch3
    p = jnp.where(owned, p_ref[...], 0.0)
    q = jnp.sum(p, axis=1)
    sx = jnp.sum(q[:, 0:16], axis=1, keepdims=True)
    sy = jnp.sum(q[:, 16:32], axis=1, keepdims=True)
    sz = jnp.sum(q[:, 32:48], axis=1, keepdims=True)
    S = jnp.concatenate([sx, sy, sz], axis=1)
    base = S[0:NS, :]
    rest = S[NS:, :]
    tiled = jnp.concatenate([base] * NOPS, axis=0)
    d = rest - tiled
    n2 = jnp.sum(d * d, axis=1, keepdims=True)
    o_ref[0, 0] = jnp.sum(w_ref[...] * jnp.sqrt(n2))


def kernel(fracs, natms, oprss, noprs):
    natms = natms.reshape(-1).astype(jnp.int32)
    noprs = noprs.reshape(-1).astype(jnp.int32)

    ti, tf, w32 = pl.pallas_call(
        _prep_body,
        in_specs=[
            pl.BlockSpec(memory_space=pltpu.SMEM),
            pl.BlockSpec(memory_space=pltpu.SMEM),
            pl.BlockSpec(memory_space=pltpu.SMEM),
        ],
        out_specs=(
            pl.BlockSpec(memory_space=pltpu.SMEM),
            pl.BlockSpec(memory_space=pltpu.SMEM),
            pl.BlockSpec(memory_space=pltpu.SMEM),
        ),
        out_shape=(
            jax.ShapeDtypeStruct((NU, 16), jnp.int32),
            jax.ShapeDtypeStruct((NU, 16), jnp.float32),
            jax.ShapeDtypeStruct((NS * NOPS, 1), jnp.float32),
        ),
    )(natms, noprs, oprss)

    mesh = plsc.VectorSubcoreMesh(core_axis_name="c", subcore_axis_name="s")
    partial = pl.kernel(
        _sc_body,
        out_type=jax.ShapeDtypeStruct((NU, NW, 48), jnp.float32),
        mesh=mesh,
        scratch_types=[
            pltpu.VMEM((3 * NTOT,), jnp.float32),
            pltpu.VMEM((3 * NTOT,), jnp.float32),
            pltpu.VMEM((NU * 16,), jnp.int32),
            pltpu.VMEM((NU * 16,), jnp.float32),
            pltpu.VMEM((48,), jnp.float32),
        ],
    )(fracs.T.reshape(-1), ti.reshape(-1), tf.reshape(-1))

    out = pl.pallas_call(
        _combine_body,
        out_specs=pl.BlockSpec(memory_space=pltpu.SMEM),
        out_shape=jax.ShapeDtypeStruct((1, 1), jnp.float32),
    )(partial, ti.reshape(NU, 16, 1), w32)
    return out[0, 0]


# speedup vs baseline: 2.4287x; 1.0866x over previous
import jax
import jax.numpy as jnp
from jax import lax
from jax.experimental import pallas as pl
from jax.experimental.pallas import tpu as pltpu
from jax.experimental.pallas import tpu_sc as plsc

NTOT = 1536
NATM = 384
NOPS = 8
NS = 4
NU = NS * (NOPS + 1)
NW = 32
R2 = 0.4 * 0.4


def _floor(x):
    t = x.astype(jnp.int32).astype(jnp.float32)
    return jnp.where(x < t, t - 1.0, t)


def _rb16(x):
    t1 = x * jnp.float32(65537.0)
    return t1 - (t1 - x)


def _sc_body(xs_hbm, ti_hbm, tf_hbm, out_hbm, xs_v, f1_v, ti_v, tf_v, st_v):
    cid = lax.axis_index("c")
    sid = lax.axis_index("s")
    w = sid * 2 + cid
    pltpu.sync_copy(xs_hbm, xs_v)
    pltpu.sync_copy(ti_hbm, ti_v)
    pltpu.sync_copy(tf_hbm, tf_v)
    lane = lax.broadcasted_iota(jnp.int32, (16,), 0)
    zero16 = jnp.zeros((16,), jnp.float32)

    def unit_body(u, k):
        tiu = ti_v[pl.ds(u * 16, 16)]
        natm = tiu[0]
        fa = tiu[1]
        ca = tiu[2]
        nch = tiu[3]
        ci = jnp.remainder(w - k, NW)

        @pl.when(ci < nch)
        def _():
            tfu = _rb16(tf_v[pl.ds(u * 16, 16)])

            @pl.when(u < NS)
            def _():
                def stage0(ch, carry):
                    gb = (ca + ch) * 16
                    for c in range(3):
                        f1_v[pl.ds(c * NTOT + gb, 16)] = \
                            xs_v[pl.ds(c * NTOT + gb, 16)]
                    return carry

                lax.fori_loop(0, nch, stage0, 0)

            @pl.when(u >= NS)
            def _():
                def stage1(ch, carry):
                    gb = (ca + ch) * 16
                    x = _rb16(xs_v[pl.ds(0 * NTOT + gb, 16)])
                    y = _rb16(xs_v[pl.ds(1 * NTOT + gb, 16)])
                    z = _rb16(xs_v[pl.ds(2 * NTOT + gb, 16)])
                    for c in range(3):
                        t = tfu[3 * c + 0] * x + tfu[3 * c + 1] * y \
                            + tfu[3 * c + 2] * z
                        f1_v[pl.ds(c * NTOT + gb, 16)] = t - _floor(t)
                    return carry

                lax.fori_loop(0, nch, stage1, 0)

            ib = (ca + ci) * 16
            ux16 = f1_v[pl.ds(0 * NTOT + ib, 16)]
            uy16 = f1_v[pl.ds(1 * NTOT + ib, 16)]
            uz16 = f1_v[pl.ds(2 * NTOT + ib, 16)]
            acc1 = (zero16, zero16, zero16)
            acc2 = (zero16, zero16, zero16)
            for g in range(2):
                rows = []
                for l in range(8):
                    li = g * 8 + l
                    gi = ib + li
                    ok_row = (gi >= fa) & (gi < fa + natm)
                    lim = jnp.where(ok_row, natm, 0).astype(jnp.uint32)
                    rows.append((jnp.broadcast_to(ux16[li], (16,)),
                                 jnp.broadcast_to(uy16[li], (16,)),
                                 jnp.broadcast_to(uz16[li], (16,)),
                                 lim))

                def cj_body(cj, accs2):
                    ax, ay, az = accs2
                    jb = (ca + cj) * 16
                    jd = (jb + lane - fa).astype(jnp.uint32)
                    xj = f1_v[pl.ds(0 * NTOT + jb, 16)]
                    yj = f1_v[pl.ds(1 * NTOT + jb, 16)]
                    zj = f1_v[pl.ds(2 * NTOT + jb, 16)]
                    for ux, uy, uz, lim in rows:
                        adx = jnp.abs(xj - ux)
                        ady = jnp.abs(yj - uy)
                        adz = jnp.abs(zj - uz)
                        tx = jnp.minimum(adx, 1.0 - adx)
                        ty = jnp.minimum(ady, 1.0 - ady)
                        tz = jnp.minimum(adz, 1.0 - adz)
                        sx = tx * tx
                        sy = ty * ty
                        sz = tz * tz
                        d2 = sx + sy + sz
                        ok = (d2 <= R2) & (jd < lim)
                        ax = ax + jnp.where(ok, sx, zero16)
                        ay = ay + jnp.where(ok, sy, zero16)
                        az = az + jnp.where(ok, sz, zero16)
                    return ax, ay, az

                acc1 = cj_body(ci, acc1)
                acc2 = lax.fori_loop(ci + 1, nch, cj_body, acc2)
            ax = acc1[0] + (acc2[0] + acc2[0])
            ay = acc1[1] + (acc2[1] + acc2[1])
            az = acc1[2] + (acc2[2] + acc2[2])
            st_v[pl.ds(0, 16)] = ax
            st_v[pl.ds(16, 16)] = ay
            st_v[pl.ds(32, 16)] = az
            pltpu.sync_copy(st_v, out_hbm.at[u, w])

        return k + nch

    lax.fori_loop(0, NU, unit_body, jnp.int32(0))


def _prep_body(natm_ref, nopr_ref, ops_ref, ti_ref, tf_ref, w_ref):
    natms = [natm_ref[i] for i in range(NS)]
    noprs = [nopr_ref[i] for i in range(NS)]
    fa = []
    oa = []
    run_a = jnp.int32(0)
    run_o = jnp.int32(0)
    for s in range(NS):
        fa.append(run_a)
        oa.append(run_o)
        run_a = run_a + natms[s]
        run_o = run_o + noprs[s]
    k_run = jnp.int32(0)
    for u in range(NU):
        s = u % NS
        slot = u // NS
        if slot == 0:
            active = jnp.bool_(True)
            for a in range(3):
                for b in range(3):
                    tf_ref[u, 3 * a + b] = jnp.float32(1.0 if a == b else 0.0)
        else:
            active = (slot - 1) < noprs[s]
            opidx = jnp.clip(oa[s] + (slot - 1), 0, 31)
            for a in range(3):
                for b in range(3):
                    tf_ref[u, 3 * a + b] = ops_ref[opidx, a, b]
        natm_eff = jnp.where(active, natms[s], 0)
        ca = fa[s] // 16
        nch = jnp.where(natm_eff > 0,
                        (fa[s] + natm_eff - 1) // 16 - ca + 1, 0)
        ti_ref[u, 0] = natm_eff
        ti_ref[u, 1] = fa[s]
        ti_ref[u, 2] = ca
        ti_ref[u, 3] = nch
        ti_ref[u, 4] = k_run
        k_run = k_run + nch
        k_run = jnp.where(k_run >= NW, k_run - NW, k_run)
    for j in range(NOPS):
        for s in range(NS):
            inv = 1.0 / (jnp.maximum(noprs[s], 1).astype(jnp.float32) * NS)
            w_ref[j * NS + s, 0] = jnp.where(j < noprs[s], inv,
                                             jnp.float32(0.0))


def _combine_body(p_ref, t3_ref, w_ref, o_ref):
    nch3 = t3_ref[:, 3:4, :]
    lo3 = t3_ref[:, 4:5, :]
    ww3 = jax.lax.broadcasted_iota(jnp.int32, (NU, NW, 1), 1)
    ci = ww3 - lo3
    ci = jnp.where(ci < 0, ci + NW, ci)
    owned = ci < nch3
    p = jnp.where(owned, p_ref[...], 0.0)
    q = jnp.sum(p, axis=1)
    sx = jnp.sum(q[:, 0:16], axis=1, keepdims=True)
    sy = jnp.sum(q[:, 16:32], axis=1, keepdims=True)
    sz = jnp.sum(q[:, 32:48], axis=1, keepdims=True)
    S = jnp.concatenate([sx, sy, sz], axis=1)
    base = S[0:NS, :]
    rest = S[NS:, :]
    tiled = jnp.concatenate([base] * NOPS, axis=0)
    d = rest - tiled
    n2 = jnp.sum(d * d, axis=1, keepdims=True)
    o_ref[0, 0] = jnp.sum(w_ref[...] * jnp.sqrt(n2))


def kernel(fracs, natms, oprss, noprs):
    natms = natms.reshape(-1).astype(jnp.int32)
    noprs = noprs.reshape(-1).astype(jnp.int32)

    ti, tf, w32 = pl.pallas_call(
        _prep_body,
        in_specs=[
            pl.BlockSpec(memory_space=pltpu.SMEM),
            pl.BlockSpec(memory_space=pltpu.SMEM),
            pl.BlockSpec(memory_space=pltpu.SMEM),
        ],
        out_specs=(
            pl.BlockSpec(memory_space=pltpu.SMEM),
            pl.BlockSpec(memory_space=pltpu.SMEM),
            pl.BlockSpec(memory_space=pltpu.SMEM),
        ),
        out_shape=(
            jax.ShapeDtypeStruct((NU, 16), jnp.int32),
            jax.ShapeDtypeStruct((NU, 16), jnp.float32),
            jax.ShapeDtypeStruct((NS * NOPS, 1), jnp.float32),
        ),
    )(natms, noprs, oprss)

    mesh = plsc.VectorSubcoreMesh(core_axis_name="c", subcore_axis_name="s")
    partial = pl.kernel(
        _sc_body,
        out_type=jax.ShapeDtypeStruct((NU, NW, 48), jnp.float32),
        mesh=mesh,
        scratch_types=[
            pltpu.VMEM((3 * NTOT,), jnp.float32),
            pltpu.VMEM((3 * NTOT,), jnp.float32),
            pltpu.VMEM((NU * 16,), jnp.int32),
            pltpu.VMEM((NU * 16,), jnp.float32),
            pltpu.VMEM((48,), jnp.float32),
        ],
    )(fracs.T.reshape(-1), ti.reshape(-1), tf.reshape(-1))

    out = pl.pallas_call(
        _combine_body,
        out_specs=pl.BlockSpec(memory_space=pltpu.SMEM),
        out_shape=jax.ShapeDtypeStruct((1, 1), jnp.float32),
    )(partial, ti.reshape(NU, 16, 1), w32)
    return out[0, 0]


# speedup vs baseline: 2.5572x; 1.0529x over previous
import jax
import jax.numpy as jnp
from jax import lax
from jax.experimental import pallas as pl
from jax.experimental.pallas import tpu as pltpu
from jax.experimental.pallas import tpu_sc as plsc

NTOT = 1536
NATM = 384
NOPS = 8
NS = 4
NU = NS * (NOPS + 1)
NW = 32
R2 = 0.4 * 0.4


def _floor(x):
    t = x.astype(jnp.int32).astype(jnp.float32)
    return jnp.where(x < t, t - 1.0, t)


def _rb16(x):
    t1 = x * jnp.float32(65537.0)
    return t1 - (t1 - x)


def _sc_body(xs_hbm, ti_hbm, tf_hbm, out_hbm, xs_v, f1_v, ti_v, tf_v, st_v):
    cid = lax.axis_index("c")
    sid = lax.axis_index("s")
    w = sid * 2 + cid
    pltpu.sync_copy(xs_hbm, xs_v)
    pltpu.sync_copy(ti_hbm, ti_v)
    pltpu.sync_copy(tf_hbm, tf_v)
    lane = lax.broadcasted_iota(jnp.int32, (16,), 0)
    zero16 = jnp.zeros((16,), jnp.float32)

    def unit_body(u, k):
        tiu = ti_v[pl.ds(u * 16, 16)]
        natm = tiu[0]
        fa = tiu[1]
        ca = tiu[2]
        nch = tiu[3]
        ci = jnp.remainder(w - k, NW)

        @pl.when(ci < nch)
        def _():
            tfu = _rb16(tf_v[pl.ds(u * 16, 16)])

            @pl.when(u < NS)
            def _():
                def stage0(ch, carry):
                    gb = (ca + ch) * 16
                    for c in range(3):
                        f1_v[pl.ds(c * NTOT + gb, 16)] = \
                            xs_v[pl.ds(c * NTOT + gb, 16)]
                    return carry

                lax.fori_loop(ci, nch, stage0, 0)

            @pl.when(u >= NS)
            def _():
                def stage1(ch, carry):
                    gb = (ca + ch) * 16
                    x = _rb16(xs_v[pl.ds(0 * NTOT + gb, 16)])
                    y = _rb16(xs_v[pl.ds(1 * NTOT + gb, 16)])
                    z = _rb16(xs_v[pl.ds(2 * NTOT + gb, 16)])
                    for c in range(3):
                        t = tfu[3 * c + 0] * x + tfu[3 * c + 1] * y \
                            + tfu[3 * c + 2] * z
                        f1_v[pl.ds(c * NTOT + gb, 16)] = t - _floor(t)
                    return carry

                lax.fori_loop(ci, nch, stage1, 0)

            ib = (ca + ci) * 16
            ux16 = f1_v[pl.ds(0 * NTOT + ib, 16)]
            uy16 = f1_v[pl.ds(1 * NTOT + ib, 16)]
            uz16 = f1_v[pl.ds(2 * NTOT + ib, 16)]
            acc1 = (zero16, zero16, zero16)
            acc2 = (zero16, zero16, zero16)
            for g in range(2):
                rows = []
                for l in range(8):
                    li = g * 8 + l
                    gi = ib + li
                    ok_row = (gi >= fa) & (gi < fa + natm)
                    lim = jnp.where(ok_row, natm, 0).astype(jnp.uint32)
                    rows.append((jnp.broadcast_to(ux16[li], (16,)),
                                 jnp.broadcast_to(uy16[li], (16,)),
                                 jnp.broadcast_to(uz16[li], (16,)),
                                 lim))

                def cj_body(cj, accs2):
                    ax, ay, az = accs2
                    jb = (ca + cj) * 16
                    jd = (jb + lane - fa).astype(jnp.uint32)
                    xj = f1_v[pl.ds(0 * NTOT + jb, 16)]
                    yj = f1_v[pl.ds(1 * NTOT + jb, 16)]
                    zj = f1_v[pl.ds(2 * NTOT + jb, 16)]
                    for ux, uy, uz, lim in rows:
                        adx = jnp.abs(xj - ux)
                        ady = jnp.abs(yj - uy)
                        adz = jnp.abs(zj - uz)
                        tx = jnp.minimum(adx, 1.0 - adx)
                        ty = jnp.minimum(ady, 1.0 - ady)
                        tz = jnp.minimum(adz, 1.0 - adz)
                        sx = tx * tx
                        sy = ty * ty
                        sz = tz * tz
                        d2 = sx + sy + sz
                        ok = (d2 <= R2) & (jd < lim)
                        ax = ax + jnp.where(ok, sx, zero16)
                        ay = ay + jnp.where(ok, sy, zero16)
                        az = az + jnp.where(ok, sz, zero16)
                    return ax, ay, az

                acc1 = cj_body(ci, acc1)
                acc2 = lax.fori_loop(ci + 1, nch, cj_body, acc2)
            ax = acc1[0] + (acc2[0] + acc2[0])
            ay = acc1[1] + (acc2[1] + acc2[1])
            az = acc1[2] + (acc2[2] + acc2[2])
            st_v[pl.ds(0, 16)] = ax
            st_v[pl.ds(16, 16)] = ay
            st_v[pl.ds(32, 16)] = az
            pltpu.sync_copy(st_v, out_hbm.at[u, w])

        return k + nch

    lax.fori_loop(0, NU, unit_body, jnp.int32(0))


def _prep_body(natm_ref, nopr_ref, ops_ref, ti_ref, tf_ref, w_ref, tn_ref):
    natms = [natm_ref[i] for i in range(NS)]
    noprs = [nopr_ref[i] for i in range(NS)]
    fa = []
    oa = []
    run_a = jnp.int32(0)
    run_o = jnp.int32(0)
    for s in range(NS):
        fa.append(run_a)
        oa.append(run_o)
        run_a = run_a + natms[s]
        run_o = run_o + noprs[s]
    k_run = jnp.int32(0)
    for u in range(NU):
        s = u % NS
        slot = u // NS
        if slot == 0:
            active = jnp.bool_(True)
            for a in range(3):
                for b in range(3):
                    tf_ref[u * 16 + 3 * a + b] = \
                        jnp.float32(1.0 if a == b else 0.0)
        else:
            active = (slot - 1) < noprs[s]
            opidx = jnp.minimum(oa[s] + (slot - 1), 31)
            for a in range(3):
                for b in range(3):
                    tf_ref[u * 16 + 3 * a + b] = ops_ref[opidx, a, b]
        natm_eff = jnp.where(active, natms[s], 0)
        ca = fa[s] // 16
        nch = jnp.where(natm_eff > 0,
                        (fa[s] + natm_eff - 1) // 16 - ca + 1, 0)
        ti_ref[u * 16 + 0] = natm_eff
        ti_ref[u * 16 + 1] = fa[s]
        ti_ref[u * 16 + 2] = ca
        ti_ref[u * 16 + 3] = nch
        tn_ref[u, 0, 0] = nch
        tn_ref[u, 1, 0] = k_run
        k_run = k_run + nch
        k_run = jnp.where(k_run >= NW, k_run - NW, k_run)
    for j in range(NOPS):
        for s in range(NS):
            inv = 1.0 / (jnp.maximum(noprs[s], 1).astype(jnp.float32) * NS)
            w_ref[j * NS + s, 0] = jnp.where(j < noprs[s], inv,
                                             jnp.float32(0.0))


def _combine_body(p_ref, t3_ref, w_ref, o_ref):
    nch3 = t3_ref[:, 0:1, :]
    lo3 = t3_ref[:, 1:2, :]
    ww3 = jax.lax.broadcasted_iota(jnp.int32, (NU, NW, 1), 1)
    ci = ww3 - lo3
    ci = jnp.where(ci < 0, ci + NW, ci)
    owned = ci < nch3
    p = jnp.where(owned, p_ref[...], 0.0)
    q = jnp.sum(p, axis=1)
    sx = jnp.sum(q[:, 0:16], axis=1, keepdims=True)
    sy = jnp.sum(q[:, 16:32], axis=1, keepdims=True)
    sz = jnp.sum(q[:, 32:48], axis=1, keepdims=True)
    S = jnp.concatenate([sx, sy, sz], axis=1)
    base = S[0:NS, :]
    rest = S[NS:, :]
    tiled = jnp.concatenate([base] * NOPS, axis=0)
    d = rest - tiled
    n2 = jnp.sum(d * d, axis=1, keepdims=True)
    o_ref[0, 0] = jnp.sum(w_ref[...] * jnp.sqrt(n2))


def kernel(fracs, natms, oprss, noprs):
    natms = natms.reshape(-1).astype(jnp.int32)
    noprs = noprs.reshape(-1).astype(jnp.int32)

    ti, tf, w32, tn = pl.pallas_call(
        _prep_body,
        in_specs=[
            pl.BlockSpec(memory_space=pltpu.SMEM),
            pl.BlockSpec(memory_space=pltpu.SMEM),
            pl.BlockSpec(memory_space=pltpu.SMEM),
        ],
        out_specs=(
            pl.BlockSpec(memory_space=pltpu.SMEM),
            pl.BlockSpec(memory_space=pltpu.SMEM),
            pl.BlockSpec(memory_space=pltpu.SMEM),
            pl.BlockSpec(memory_space=pltpu.SMEM),
        ),
        out_shape=(
            jax.ShapeDtypeStruct((NU * 16,), jnp.int32),
            jax.ShapeDtypeStruct((NU * 16,), jnp.float32),
            jax.ShapeDtypeStruct((NS * NOPS, 1), jnp.float32),
            jax.ShapeDtypeStruct((NU, 2, 1), jnp.int32),
        ),
    )(natms, noprs, oprss)

    mesh = plsc.VectorSubcoreMesh(core_axis_name="c", subcore_axis_name="s")
    partial = pl.kernel(
        _sc_body,
        out_type=jax.ShapeDtypeStruct((NU, NW, 48), jnp.float32),
        mesh=mesh,
        scratch_types=[
            pltpu.VMEM((3 * NTOT,), jnp.float32),
            pltpu.VMEM((3 * NTOT,), jnp.float32),
            pltpu.VMEM((NU * 16,), jnp.int32),
            pltpu.VMEM((NU * 16,), jnp.float32),
            pltpu.VMEM((48,), jnp.float32),
        ],
    )(fracs.T.reshape(-1), ti, tf)

    out = pl.pallas_call(
        _combine_body,
        out_specs=pl.BlockSpec(memory_space=pltpu.SMEM),
        out_shape=jax.ShapeDtypeStruct((1, 1), jnp.float32),
    )(partial, tn, w32)
    return out[0, 0]
